# baseline (speedup 1.0000x reference)
import functools

import jax
import jax.numpy as jnp
from jax.experimental import pallas as pl
from jax.experimental.pallas import tpu as pltpu

_TM = 256


def _silu(v):
    return v * jax.nn.sigmoid(v)


def _moe_body(x_ref, Wg_ref, bg_ref, Weg_ref, beg_ref, Weu_ref, beu_ref,
              Wdn_ref, bed_ref, Wsu_ref, bsu_ref, Wsd_ref, bsd_ref, out_ref,
              *, n_routed):
    x = x_ref[...]
    xb = x.astype(jnp.bfloat16)

    gate = jnp.dot(x, Wg_ref[...], preferred_element_type=jnp.float32)
    gate = gate + bg_ref[...]
    idx = jax.lax.broadcasted_iota(jnp.int32, gate.shape, 1)
    m1 = jnp.min(gate, axis=-1, keepdims=True)
    e1 = jnp.max(jnp.where(gate == m1, idx, -1), axis=-1, keepdims=True)
    g2 = jnp.where(idx == e1, jnp.inf, gate)
    m2 = jnp.min(g2, axis=-1, keepdims=True)
    e2 = jnp.max(jnp.where(g2 == m2, idx, -1), axis=-1, keepdims=True)
    excluded = (idx == e1) | (idx == e2)
    mx = jnp.max(gate, axis=-1, keepdims=True)
    exv = jnp.where(excluded, 0.0, jnp.exp(gate - mx))
    p = exv / jnp.sum(exv, axis=-1, keepdims=True)

    blocks = []
    for i in range(n_routed):
        g = jnp.dot(xb, Weg_ref[i], preferred_element_type=jnp.float32)
        u = jnp.dot(xb, Weu_ref[i], preferred_element_type=jnp.float32)
        g = g + beg_ref[i:i + 1]
        u = u + beu_ref[i:i + 1]
        blocks.append((_silu(g) * u * p[:, i:i + 1]).astype(jnp.bfloat16))
    H2 = jnp.concatenate(blocks, axis=1)

    s = jnp.dot(xb, Wsu_ref[...], preferred_element_type=jnp.float32)
    a = _silu(s + bsu_ref[...]).astype(jnp.bfloat16)

    acc = jnp.dot(H2, Wdn_ref[...], preferred_element_type=jnp.float32)
    acc = acc + jnp.dot(a, Wsd_ref[...], preferred_element_type=jnp.float32)
    acc = acc + bsd_ref[...]
    acc = acc + jnp.dot(p[:, :n_routed], bed_ref[...],
                        preferred_element_type=jnp.float32)
    out_ref[...] = acc


def _whole(shape):
    nd = len(shape)
    return pl.BlockSpec(shape, lambda i: (0,) * nd)


@jax.jit
def kernel(x, Wg, bg, We_gate, be_gate, We_up, be_up, We_down, be_down,
           Wsu, bsu, Wsd, bsd):
    B, S, H = x.shape
    T = B * S
    E = Wg.shape[1]
    n_routed, _, ex = We_gate.shape
    nex = n_routed * ex
    xf = x.reshape(T, H)

    bf = jnp.bfloat16
    Wegb = We_gate.astype(bf)
    Weub = We_up.astype(bf)
    Wdnb = We_down.astype(bf).reshape(nex, H)
    Wsub = Wsu.astype(bf)
    Wsdb = Wsd.astype(bf)
    bg2 = bg.reshape(1, E)
    bsu2 = bsu.reshape(1, -1)
    bsd2 = bsd.reshape(1, H)

    body = functools.partial(_moe_body, n_routed=n_routed)

    out = pl.pallas_call(
        body,
        grid=(T // _TM,),
        in_specs=[
            pl.BlockSpec((_TM, H), lambda i: (i, 0)),
            _whole(Wg.shape),
            _whole(bg2.shape),
            _whole(Wegb.shape),
            _whole(be_gate.shape),
            _whole(Weub.shape),
            _whole(be_up.shape),
            _whole(Wdnb.shape),
            _whole(be_down.shape),
            _whole(Wsub.shape),
            _whole(bsu2.shape),
            _whole(Wsdb.shape),
            _whole(bsd2.shape),
        ],
        out_specs=pl.BlockSpec((_TM, H), lambda i: (i, 0)),
        out_shape=jax.ShapeDtypeStruct((T, H), jnp.float32),
    )(xf, Wg, bg2, Wegb, be_gate, Weub, be_up, Wdnb, be_down,
      Wsub, bsu2, Wsdb, bsd2)
    return out.reshape(B, S, H)
